# baseline (speedup 1.0000x reference)
import functools

import jax
import jax.numpy as jnp
import numpy as np
from jax import lax
from jax.experimental import pallas as pl
from jax.experimental.pallas import tpu as pltpu
from jax.experimental.pallas import tpu_sc as plsc

_V_MIN = -10.0
_V_MAX = 10.0
_NUM_ATOMS = 51
_DISCOUNT = 0.99
_ATOM_DELTA = (_V_MAX - _V_MIN) / (_NUM_ATOMS - 1)
_ATOMS_F32 = np.asarray(
    [_V_MIN + _ATOM_DELTA * i for i in range(_NUM_ATOMS)], dtype=np.float32
)

_NC = 2
_NS = 16
_L = 16
_NW = _NC * _NS


@functools.lru_cache(maxsize=None)
def _make_kernel(bs: int, num_atoms: int):
    A = num_atoms
    cols_per_w = bs // _NW
    chunk = min(256, cols_per_w)
    n_chunks = cols_per_w // chunk
    groups = chunk // _L
    assert n_chunks % 2 == 0

    mesh = plsc.VectorSubcoreMesh(
        core_axis_name="c", subcore_axis_name="s",
        num_cores=_NC, num_subcores=_NS,
    )

    @functools.partial(
        pl.kernel,
        out_type=jax.ShapeDtypeStruct((A, bs), jnp.float32),
        mesh=mesh,
        compiler_params=pltpu.CompilerParams(
            needs_layout_passes=False, use_tc_tiling_on_sc=True),
        scratch_types=[
            pltpu.VMEM((cols_per_w,), jnp.float32),
            pltpu.VMEM((cols_per_w,), jnp.float32),
            pltpu.VMEM((A, chunk), jnp.float32),
            pltpu.VMEM((A, chunk), jnp.float32),
            pltpu.VMEM((A, chunk), jnp.float32),
            pltpu.VMEM((A, chunk), jnp.float32),
            pltpu.VMEM(((A + 1) * chunk,), jnp.float32),
            pltpu.SemaphoreType.DMA,
            pltpu.SemaphoreType.DMA,
            pltpu.SemaphoreType.DMA,
            pltpu.SemaphoreType.DMA,
        ],
    )
    def projection_kernel(rew_hbm, nd_hbm, probs_hbm, out_hbm,
                          rew_v, nd_v, si0, si1, so0, so1,
                          acc_l, isem0, isem1, osem0, osem1):
        cid = lax.axis_index("c")
        sid = lax.axis_index("s")
        wid = sid * _NC + cid
        col0 = wid * cols_per_w
        pltpu.sync_copy(rew_hbm.at[pl.ds(col0, cols_per_w)], rew_v)
        pltpu.sync_copy(nd_hbm.at[pl.ds(col0, cols_per_w)], nd_v)
        lanes = lax.iota(jnp.int32, _L)
        zeros = jnp.zeros((_L,), jnp.float32)
        stages_in = (si0, si1)
        stages_out = (so0, so1)
        isems = (isem0, isem1)
        osems = (osem0, osem1)

        def in_cols(t):
            return probs_hbm.at[:, pl.ds(col0 + t * chunk, chunk)]

        def out_cols(t):
            return out_hbm.at[:, pl.ds(col0 + t * chunk, chunk)]

        pltpu.async_copy(in_cols(0), si0, isem0)
        pltpu.async_copy(in_cols(1), si1, isem1)

        def super_body(tt, carry):
            for bsel in range(2):
                t = tt * 2 + bsel
                s_in = stages_in[bsel]
                s_out = stages_out[bsel]
                pltpu.make_async_copy(in_cols(t), s_in, isems[bsel]).wait()
                cbase = t * chunk

                @plsc.parallel_loop(0, groups)
                def group_body(g):
                    c0 = g * _L
                    r = rew_v[pl.ds(cbase + c0, _L)]
                    nd = nd_v[pl.ds(cbase + c0, _L)]
                    c = _DISCOUNT * nd
                    colv = c0 + lanes
                    for kk in range(A):
                        acc_l[pl.ds(kk * chunk + c0, _L)] = zeros
                    for j in range(A):
                        pj = s_in[j, pl.ds(c0, _L)]
                        z = r + c * float(_ATOMS_F32[j])
                        z = jnp.maximum(z, _V_MIN)
                        z = jnp.minimum(z, _V_MAX)
                        bb = (z - _V_MIN) / _ATOM_DELTA
                        li = bb.astype(jnp.int32)
                        frac = bb - li.astype(jnp.float32)
                        wu = frac * pj
                        wl = pj - wu
                        idxl = li * chunk + colv
                        idxu = idxl + chunk
                        plsc.addupdate_scatter(acc_l, [idxl], wl)
                        plsc.addupdate_scatter(acc_l, [idxu], wu)

                @pl.when(t + 2 < n_chunks)
                def _():
                    pltpu.async_copy(in_cols(t + 2), s_in, isems[bsel])

                @pl.when(t >= 2)
                def _():
                    pltpu.make_async_copy(s_out, out_cols(t - 2),
                                          osems[bsel]).wait()

                @plsc.parallel_loop(0, A, unroll=2)
                def repack_out(a):
                    base = a * chunk
                    for cc in range(groups):
                        s_out[a, pl.ds(cc * _L, _L)] = (
                            acc_l[pl.ds(base + cc * _L, _L)])

                pltpu.async_copy(s_out, out_cols(t), osems[bsel])
            return carry

        lax.fori_loop(0, n_chunks // 2, super_body, 0)
        pltpu.make_async_copy(so0, out_cols(n_chunks - 2), osem0).wait()
        pltpu.make_async_copy(so1, out_cols(n_chunks - 1), osem1).wait()

    return projection_kernel


def kernel(reward, probs, not_done):
    bs, A = probs.shape
    run = _make_kernel(bs, A)
    out_t = run(reward.reshape(bs), not_done.reshape(bs), probs.T)
    return out_t.T


# speedup vs baseline: 1.1307x; 1.1307x over previous
import functools

import jax
import jax.numpy as jnp
import numpy as np
from jax import lax
from jax.experimental import pallas as pl
from jax.experimental.pallas import tpu as pltpu
from jax.experimental.pallas import tpu_sc as plsc

_V_MIN = -10.0
_V_MAX = 10.0
_NUM_ATOMS = 51
_DISCOUNT = 0.99
_ATOM_DELTA = (_V_MAX - _V_MIN) / (_NUM_ATOMS - 1)
_ATOMS_F32 = np.asarray(
    [_V_MIN + _ATOM_DELTA * i for i in range(_NUM_ATOMS)], dtype=np.float32
)

_NC = 2
_NS = 16
_L = 16
_NW = _NC * _NS


@functools.lru_cache(maxsize=None)
def _make_kernel(bs: int, num_atoms: int):
    A = num_atoms
    cols_per_w = bs // _NW
    chunk = min(256, cols_per_w)
    n_chunks = cols_per_w // chunk
    groups = chunk // _L
    assert n_chunks % 2 == 0

    mesh = plsc.VectorSubcoreMesh(
        core_axis_name="c", subcore_axis_name="s",
        num_cores=_NC, num_subcores=_NS,
    )

    @functools.partial(
        pl.kernel,
        out_type=jax.ShapeDtypeStruct((A, bs), jnp.float32),
        mesh=mesh,
        compiler_params=pltpu.CompilerParams(
            needs_layout_passes=False, use_tc_tiling_on_sc=True),
        scratch_types=[
            pltpu.VMEM((cols_per_w,), jnp.float32),
            pltpu.VMEM((cols_per_w,), jnp.float32),
            pltpu.VMEM((A, chunk), jnp.float32),
            pltpu.VMEM((A, chunk), jnp.float32),
            pltpu.VMEM((A, chunk), jnp.float32),
            pltpu.VMEM((A, chunk), jnp.float32),
            pltpu.VMEM(((A + 1) * chunk,), jnp.float32),
            pltpu.SemaphoreType.DMA,
            pltpu.SemaphoreType.DMA,
            pltpu.SemaphoreType.DMA,
            pltpu.SemaphoreType.DMA,
        ],
    )
    def projection_kernel(rew_hbm, nd_hbm, probs_hbm, out_hbm,
                          rew_v, nd_v, si0, si1, so0, so1,
                          acc_l, isem0, isem1, osem0, osem1):
        cid = lax.axis_index("c")
        sid = lax.axis_index("s")
        wid = sid * _NC + cid
        col0 = wid * cols_per_w
        pltpu.sync_copy(rew_hbm.at[pl.ds(col0, cols_per_w)], rew_v)
        pltpu.sync_copy(nd_hbm.at[pl.ds(col0, cols_per_w)], nd_v)
        lanes = lax.iota(jnp.int32, _L)
        zeros = jnp.zeros((_L,), jnp.float32)
        stages_in = (si0, si1)
        stages_out = (so0, so1)
        isems = (isem0, isem1)
        osems = (osem0, osem1)

        def in_cols(t):
            return probs_hbm.at[:, pl.ds(col0 + t * chunk, chunk)]

        def out_cols(t):
            return out_hbm.at[:, pl.ds(col0 + t * chunk, chunk)]

        pltpu.async_copy(in_cols(0), si0, isem0)
        pltpu.async_copy(in_cols(1), si1, isem1)

        def super_body(tt, carry):
            for bsel in range(2):
                t = tt * 2 + bsel
                s_in = stages_in[bsel]
                s_out = stages_out[bsel]
                pltpu.make_async_copy(in_cols(t), s_in, isems[bsel]).wait()
                cbase = t * chunk

                @plsc.parallel_loop(0, groups)
                def group_body(g):
                    c0 = g * _L
                    r = rew_v[pl.ds(cbase + c0, _L)]
                    nd = nd_v[pl.ds(cbase + c0, _L)]
                    c = _DISCOUNT * nd
                    colv = c0 + lanes
                    for kk in range(A):
                        acc_l[pl.ds(kk * chunk + c0, _L)] = zeros
                    for j in range(A):
                        pj = s_in[j, pl.ds(c0, _L)]
                        z = r + c * float(_ATOMS_F32[j])
                        z = jnp.maximum(z, _V_MIN)
                        z = jnp.minimum(z, _V_MAX)
                        bb = (z - _V_MIN) / _ATOM_DELTA
                        li = bb.astype(jnp.int32)
                        frac = bb - li.astype(jnp.float32)
                        wl = (1.0 - frac) * pj
                        wu = frac * pj
                        idxl = li * chunk + colv
                        idxu = idxl + chunk
                        plsc.addupdate_scatter(acc_l, [idxl], wl)
                        plsc.addupdate_scatter(acc_l, [idxu], wu)

                @pl.when(t + 2 < n_chunks)
                def _():
                    pltpu.async_copy(in_cols(t + 2), s_in, isems[bsel])

                @pl.when(t >= 2)
                def _():
                    pltpu.make_async_copy(s_out, out_cols(t - 2),
                                          osems[bsel]).wait()

                @plsc.parallel_loop(0, A, unroll=2)
                def repack_out(a):
                    base = a * chunk
                    for cc in range(groups):
                        s_out[a, pl.ds(cc * _L, _L)] = (
                            acc_l[pl.ds(base + cc * _L, _L)])

                pltpu.async_copy(s_out, out_cols(t), osems[bsel])
            return carry

        lax.fori_loop(0, n_chunks // 2, super_body, 0)
        pltpu.make_async_copy(so0, out_cols(n_chunks - 2), osem0).wait()
        pltpu.make_async_copy(so1, out_cols(n_chunks - 1), osem1).wait()

    return projection_kernel


def kernel(reward, probs, not_done):
    bs, A = probs.shape
    run = _make_kernel(bs, A)
    out_t = run(reward.reshape(bs), not_done.reshape(bs), probs.T)
    return out_t.T


# speedup vs baseline: 1.2309x; 1.0886x over previous
import functools

import jax
import jax.numpy as jnp
import numpy as np
from jax import lax
from jax.experimental import pallas as pl
from jax.experimental.pallas import tpu as pltpu
from jax.experimental.pallas import tpu_sc as plsc

_V_MIN = -10.0
_V_MAX = 10.0
_NUM_ATOMS = 51
_DISCOUNT = 0.99
_ATOM_DELTA = (_V_MAX - _V_MIN) / (_NUM_ATOMS - 1)
_ATOMS_F32 = np.asarray(
    [_V_MIN + _ATOM_DELTA * i for i in range(_NUM_ATOMS)], dtype=np.float32
)

_NC = 2
_NS = 16
_L = 16
_NW = _NC * _NS


@functools.lru_cache(maxsize=None)
def _make_kernel(bs: int, num_atoms: int):
    A = num_atoms
    cols_per_w = bs // _NW
    chunk = min(256, cols_per_w)
    n_chunks = cols_per_w // chunk
    groups = chunk // _L
    assert n_chunks % 2 == 0

    mesh = plsc.VectorSubcoreMesh(
        core_axis_name="c", subcore_axis_name="s",
        num_cores=_NC, num_subcores=_NS,
    )

    @functools.partial(
        pl.kernel,
        out_type=jax.ShapeDtypeStruct((A, bs), jnp.float32),
        mesh=mesh,
        compiler_params=pltpu.CompilerParams(
            needs_layout_passes=False, use_tc_tiling_on_sc=True),
        scratch_types=[
            pltpu.VMEM((cols_per_w,), jnp.float32),
            pltpu.VMEM((cols_per_w,), jnp.float32),
            pltpu.VMEM((A, chunk), jnp.float32),
            pltpu.VMEM((A, chunk), jnp.float32),
            pltpu.VMEM((A, chunk), jnp.float32),
            pltpu.VMEM((A, chunk), jnp.float32),
            pltpu.VMEM(((A + 1) * chunk,), jnp.float32),
            pltpu.SemaphoreType.DMA,
            pltpu.SemaphoreType.DMA,
            pltpu.SemaphoreType.DMA,
            pltpu.SemaphoreType.DMA,
        ],
    )
    def projection_kernel(rew_hbm, nd_hbm, probs_hbm, out_hbm,
                          rew_v, nd_v, si0, si1, so0, so1,
                          acc_l, isem0, isem1, osem0, osem1):
        cid = lax.axis_index("c")
        sid = lax.axis_index("s")
        wid = sid * _NC + cid
        col0 = wid * cols_per_w
        pltpu.sync_copy(rew_hbm.at[pl.ds(col0, cols_per_w)], rew_v)
        pltpu.sync_copy(nd_hbm.at[pl.ds(col0, cols_per_w)], nd_v)
        lanes = lax.iota(jnp.int32, _L)
        zeros = jnp.zeros((_L,), jnp.float32)
        stages_in = (si0, si1)
        stages_out = (so0, so1)
        isems = (isem0, isem1)
        osems = (osem0, osem1)

        def in_cols(t):
            return probs_hbm.at[:, pl.ds(col0 + t * chunk, chunk)]

        def out_cols(t):
            return out_hbm.at[:, pl.ds(col0 + t * chunk, chunk)]

        pltpu.async_copy(in_cols(0), si0, isem0)
        pltpu.async_copy(in_cols(1), si1, isem1)

        def super_body(tt, carry):
            for bsel in range(2):
                t = tt * 2 + bsel
                s_in = stages_in[bsel]
                s_out = stages_out[bsel]
                pltpu.make_async_copy(in_cols(t), s_in, isems[bsel]).wait()
                cbase = t * chunk

                @plsc.parallel_loop(0, groups, unroll=2)
                def group_body(g):
                    c0 = g * _L
                    r = rew_v[pl.ds(cbase + c0, _L)]
                    nd = nd_v[pl.ds(cbase + c0, _L)]
                    c = _DISCOUNT * nd
                    colv = c0 + lanes
                    for kk in range(A):
                        acc_l[pl.ds(kk * chunk + c0, _L)] = zeros
                    for j in range(A):
                        pj = s_in[j, pl.ds(c0, _L)]
                        z = r + c * float(_ATOMS_F32[j])
                        z = jnp.maximum(z, _V_MIN)
                        z = jnp.minimum(z, _V_MAX)
                        bb = (z - _V_MIN) / _ATOM_DELTA
                        li = bb.astype(jnp.int32)
                        frac = bb - li.astype(jnp.float32)
                        wl = (1.0 - frac) * pj
                        wu = frac * pj
                        idxl = li * chunk + colv
                        idxu = idxl + chunk
                        plsc.addupdate_scatter(acc_l, [idxl], wl)
                        plsc.addupdate_scatter(acc_l, [idxu], wu)

                @pl.when(t + 2 < n_chunks)
                def _():
                    pltpu.async_copy(in_cols(t + 2), s_in, isems[bsel])

                @pl.when(t >= 2)
                def _():
                    pltpu.make_async_copy(s_out, out_cols(t - 2),
                                          osems[bsel]).wait()

                @plsc.parallel_loop(0, A, unroll=2)
                def repack_out(a):
                    base = a * chunk
                    for cc in range(groups):
                        s_out[a, pl.ds(cc * _L, _L)] = (
                            acc_l[pl.ds(base + cc * _L, _L)])

                pltpu.async_copy(s_out, out_cols(t), osems[bsel])
            return carry

        lax.fori_loop(0, n_chunks // 2, super_body, 0)
        pltpu.make_async_copy(so0, out_cols(n_chunks - 2), osem0).wait()
        pltpu.make_async_copy(so1, out_cols(n_chunks - 1), osem1).wait()

    return projection_kernel


def kernel(reward, probs, not_done):
    bs, A = probs.shape
    run = _make_kernel(bs, A)
    out_t = run(reward.reshape(bs), not_done.reshape(bs), probs.T)
    return out_t.T
